# Initial kernel scaffold; baseline (speedup 1.0000x reference)
import jax
import jax.numpy as jnp
from jax.experimental import pallas as pl


def kernel(x, edge_index, edge_weight, delta, a):
    raise NotImplementedError("write your pallas kernel here")


# speedup vs baseline: 2.0075x; 2.0075x over previous
import functools

import jax
import jax.numpy as jnp
from jax import lax
from jax.experimental import pallas as pl
from jax.experimental.pallas import tpu as pltpu
from jax.experimental.pallas import tpu_sc as plsc

N_NODES = 10000
N_EDGES = 320000
D_FEAT = 128
D_HALF = 64

N_PAD = 10240
N_SUBCORES = 16
ROWS_PER_TILE = N_PAD // N_SUBCORES
EDGES_PER_TILE = N_EDGES // N_SUBCORES
CHUNK_E = 80
N_CHUNKS = EDGES_PER_TILE // CHUNK_E
ROW_CHUNK = 64
N_ROW_CHUNKS = ROWS_PER_TILE // ROW_CHUNK


def _zero_fill(ref, n_rows):
    zv = jnp.zeros((16,), jnp.float32)

    def body(j, _):
        for d in range(D_HALF // 16):
            ref[j, pl.ds(16 * d, 16)] = zv
        return _

    lax.fori_loop(0, n_rows, body, None)


def _hop(src_buf, acc_buf, srcv, dstv, wv, rows, src_hbm, dst_hbm, w_hbm, sid):
    e_base = sid * EDGES_PER_TILE

    def chunk_body(i, _):
        base = e_base + i * CHUNK_E
        pltpu.sync_copy(src_hbm.at[pl.ds(base, CHUNK_E)], srcv)
        pltpu.sync_copy(dst_hbm.at[pl.ds(base, CHUNK_E)], dstv)
        pltpu.sync_copy(w_hbm.at[pl.ds(base, CHUNK_E)], wv)
        pltpu.sync_copy(src_buf.at[srcv], rows)

        def group_body(g, _):
            wgrp = wv[pl.ds(g * 16, 16)]
            for j16 in range(16):
                j = g * 16 + j16
                wvec = jnp.full((16,), wgrp[j16], jnp.float32)
                for d in range(D_HALF // 16):
                    sl = pl.ds(16 * d, 16)
                    rows[j, sl] = rows[j, sl] * wvec
            return _

        lax.fori_loop(0, CHUNK_E // 16, group_body, None)
        pltpu.sync_copy(rows, acc_buf.at[dstv], add=True)
        return _

    lax.fori_loop(0, N_CHUNKS, chunk_body, None)


def _combine_chunk(t0c, t1c, t2c, t3c, lowc, highc, coefs):
    c2l, c1l, c0l, c2h, c1h, c0h = coefs

    def body(j, _):
        for d in range(D_HALF // 16):
            sl = pl.ds(16 * d, 16)
            v0 = t0c[j, sl]
            v1 = t1c[j, sl]
            v2 = t2c[j, sl]
            v3 = t3c[j, sl]
            lowc[j, sl] = v3 + c2l * v2 + c1l * v1 + c0l * v0
            highc[j, sl] = v3 + c2h * v2 + c1h * v1 + c0h * v0
        return _

    lax.fori_loop(0, ROW_CHUNK, body, None)


def _filter_kernel(x_pair, src_hbm, dst_hbm, w_hbm, da_hbm,
                   low_hbm, high_hbm, tx1_hbm, tx2_hbm,
                   buf_a, buf_b,
                   srcv, dstv, wv, rows, zbuf,
                   t0c, t1c, t2c, t3c, lowc, highc, dav):
    cid = lax.axis_index("c")
    sid = lax.axis_index("s")
    r0 = sid * ROWS_PER_TILE

    pltpu.sync_copy(x_pair.at[cid, pl.ds(r0, ROWS_PER_TILE)],
                    buf_a.at[pl.ds(r0, ROWS_PER_TILE)])
    pltpu.sync_copy(da_hbm, dav)
    _zero_fill(zbuf, ROW_CHUNK)
    for q in range(N_ROW_CHUNKS):
        pltpu.sync_copy(zbuf, buf_b.at[pl.ds(r0 + q * ROW_CHUNK, ROW_CHUNK)])
    plsc.subcore_barrier()

    _hop(buf_a, buf_b, srcv, dstv, wv, rows, src_hbm, dst_hbm, w_hbm, sid)
    plsc.subcore_barrier()

    pltpu.sync_copy(buf_b.at[pl.ds(r0, ROWS_PER_TILE)],
                    tx1_hbm.at[cid, pl.ds(r0, ROWS_PER_TILE)])
    for q in range(N_ROW_CHUNKS):
        pltpu.sync_copy(zbuf, buf_a.at[pl.ds(r0 + q * ROW_CHUNK, ROW_CHUNK)])
    plsc.subcore_barrier()

    _hop(buf_b, buf_a, srcv, dstv, wv, rows, src_hbm, dst_hbm, w_hbm, sid)
    plsc.subcore_barrier()

    pltpu.sync_copy(buf_a.at[pl.ds(r0, ROWS_PER_TILE)],
                    tx2_hbm.at[cid, pl.ds(r0, ROWS_PER_TILE)])
    for q in range(N_ROW_CHUNKS):
        pltpu.sync_copy(zbuf, buf_b.at[pl.ds(r0 + q * ROW_CHUNK, ROW_CHUNK)])
    plsc.subcore_barrier()

    _hop(buf_a, buf_b, srcv, dstv, wv, rows, src_hbm, dst_hbm, w_hbm, sid)
    plsc.subcore_barrier()

    davec = dav[pl.ds(0, 16)]
    d = davec[0]
    av = davec[1]
    d2 = d * d
    c2l = -3.0 * d - av
    c1l = 3.0 * d2 + 2.0 * d * av
    c0l = -(d2 * d + d2 * av)
    c2h = -3.0 * d + av
    c1h = 3.0 * d2 - 2.0 * d * av
    c0h = d2 * av - d2 * d
    coefs = (c2l, c1l, c0l, c2h, c1h, c0h)

    for q in range(N_ROW_CHUNKS):
        rq = r0 + q * ROW_CHUNK
        pltpu.sync_copy(x_pair.at[cid, pl.ds(rq, ROW_CHUNK)], t0c)
        pltpu.sync_copy(tx1_hbm.at[cid, pl.ds(rq, ROW_CHUNK)], t1c)
        pltpu.sync_copy(tx2_hbm.at[cid, pl.ds(rq, ROW_CHUNK)], t2c)
        pltpu.sync_copy(buf_b.at[pl.ds(rq, ROW_CHUNK)], t3c)
        _combine_chunk(t0c, t1c, t2c, t3c, lowc, highc, coefs)
        pltpu.sync_copy(lowc, low_hbm.at[cid, pl.ds(rq, ROW_CHUNK)])
        pltpu.sync_copy(highc, high_hbm.at[cid, pl.ds(rq, ROW_CHUNK)])


@jax.jit
def _run(x_pair, src, dst, w, da):
    mesh = plsc.VectorSubcoreMesh(core_axis_name="c", subcore_axis_name="s")
    f = pl.kernel(
        _filter_kernel,
        mesh=mesh,
        compiler_params=pltpu.CompilerParams(use_tc_tiling_on_sc=False),
        out_type=[
            jax.ShapeDtypeStruct((2, N_PAD, D_HALF), jnp.float32),
            jax.ShapeDtypeStruct((2, N_PAD, D_HALF), jnp.float32),
            jax.ShapeDtypeStruct((2, N_PAD, D_HALF), jnp.float32),
            jax.ShapeDtypeStruct((2, N_PAD, D_HALF), jnp.float32),
        ],
        scratch_types=[
            pltpu.VMEM_SHARED((N_PAD, D_HALF), jnp.float32),
            pltpu.VMEM_SHARED((N_PAD, D_HALF), jnp.float32),
            pltpu.VMEM((CHUNK_E,), jnp.int32),
            pltpu.VMEM((CHUNK_E,), jnp.int32),
            pltpu.VMEM((CHUNK_E,), jnp.float32),
            pltpu.VMEM((CHUNK_E, D_HALF), jnp.float32),
            pltpu.VMEM((ROW_CHUNK, D_HALF), jnp.float32),
            pltpu.VMEM((ROW_CHUNK, D_HALF), jnp.float32),
            pltpu.VMEM((ROW_CHUNK, D_HALF), jnp.float32),
            pltpu.VMEM((ROW_CHUNK, D_HALF), jnp.float32),
            pltpu.VMEM((ROW_CHUNK, D_HALF), jnp.float32),
            pltpu.VMEM((ROW_CHUNK, D_HALF), jnp.float32),
            pltpu.VMEM((ROW_CHUNK, D_HALF), jnp.float32),
            pltpu.VMEM((16,), jnp.float32),
        ],
    )
    return f(x_pair, src, dst, w, da)


def kernel(x, edge_index, edge_weight, delta, a):
    x_pair = x.reshape(N_NODES, 2, D_HALF).transpose(1, 0, 2)
    x_pair = jnp.pad(x_pair, ((0, 0), (0, N_PAD - N_NODES), (0, 0)))
    src = edge_index[1]
    dst = edge_index[0]
    da = jnp.concatenate([delta, a, jnp.zeros((14,), jnp.float32)])
    low_p, high_p, _tx1, _tx2 = _run(x_pair, src, dst, edge_weight, da)
    low = low_p[:, :N_NODES].transpose(1, 0, 2).reshape(N_NODES, D_FEAT)
    high = high_p[:, :N_NODES].transpose(1, 0, 2).reshape(N_NODES, D_FEAT)
    return (low, high)


# speedup vs baseline: 3.8109x; 1.8984x over previous
import jax
import jax.numpy as jnp
from jax import lax
from jax.experimental import pallas as pl
from jax.experimental.pallas import tpu as pltpu
from jax.experimental.pallas import tpu_sc as plsc

N_NODES = 10000
N_EDGES = 320000
D_FEAT = 128
D_HALF = 64

N_PAD = 10240
N_SUBCORES = 16
ROWS_PER_TILE = N_PAD // N_SUBCORES
CHUNK_E = 128
N_CHUNKS = 158
E_PAD = N_CHUNKS * CHUNK_E * N_SUBCORES
ROW_CHUNK = 32
N_ROW_CHUNKS = ROWS_PER_TILE // ROW_CHUNK


def _zero_fill(ref, n_rows):
    zv = jnp.zeros((16,), jnp.float32)

    def body(j, _):
        for d in range(D_HALF // 16):
            ref[j, pl.ds(16 * d, 16)] = zv
        return _

    lax.fori_loop(0, n_rows, body, None)


def _scale_rows(wbuf, rows):
    def group_body(g, _):
        wgrp = wbuf[pl.ds(g * 16, 16)]
        for j16 in range(16):
            j = g * 16 + j16
            wvec = jnp.full((16,), wgrp[j16], jnp.float32)
            for d in range(D_HALF // 16):
                sl = pl.ds(16 * d, 16)
                rows[j, sl] = rows[j, sl] * wvec
        return _

    lax.fori_loop(0, CHUNK_E // 16, group_body, None)


def _copy_dst(ebuf, dstc):
    for k in range(CHUNK_E // 16):
        dstc[pl.ds(16 * k, 16)] = ebuf[1, pl.ds(16 * k, 16)]


def _hop(src_buf, acc_buf, epack, wpack, sid, bufs):
    (ebuf0, ebuf1, wbuf0, wbuf1, rows0, rows1, dstc0, dstc1,
     se0, se1, sw0, sw1, sg0, sg1, ss0, ss1) = bufs
    c_base = sid * N_CHUNKS
    last_even = N_CHUNKS - 2
    last_odd = N_CHUNKS - 1

    def load_e(ebuf, sem, c):
        return pltpu.async_copy(epack.at[c_base + c], ebuf, sem)

    def load_w(wbuf, sem, c):
        return pltpu.async_copy(wpack.at[c_base + c], wbuf, sem)

    def gather(ebuf, rows, sem):
        return pltpu.async_copy(src_buf.at[ebuf.at[0]], rows, sem)

    def scatter(rows, dstc, sem):
        return pltpu.async_copy(rows, acc_buf.at[dstc], sem, add=True)

    le0 = load_e(ebuf0, se0, 0)
    le1 = load_e(ebuf1, se1, 1)
    lw0 = load_w(wbuf0, sw0, 0)
    lw1 = load_w(wbuf1, sw1, 1)
    le0.wait()
    g0 = gather(ebuf0, rows0, sg0)
    le1.wait()
    g0.wait()
    g1 = gather(ebuf1, rows1, sg1)
    lw0.wait()
    _copy_dst(ebuf0, dstc0)
    _scale_rows(wbuf0, rows0)
    load_e(ebuf0, se0, 2)
    load_w(wbuf0, sw0, 2)
    s0 = scatter(rows0, dstc0, ss0)
    g1.wait()
    lw1.wait()
    _copy_dst(ebuf1, dstc1)
    _scale_rows(wbuf1, rows1)
    load_e(ebuf1, se1, 3)
    load_w(wbuf1, sw1, 3)
    s1 = scatter(rows1, dstc1, ss1)

    def body(i, _):
        pltpu.make_async_copy(rows0, acc_buf.at[dstc0], ss0).wait()
        pltpu.make_async_copy(epack.at[c_base], ebuf0, se0).wait()
        gA = gather(ebuf0, rows0, sg0)
        pltpu.make_async_copy(rows1, acc_buf.at[dstc1], ss1).wait()
        pltpu.make_async_copy(epack.at[c_base], ebuf1, se1).wait()
        gA.wait()
        gB = gather(ebuf1, rows1, sg1)
        pltpu.make_async_copy(wpack.at[c_base], wbuf0, sw0).wait()
        _copy_dst(ebuf0, dstc0)
        _scale_rows(wbuf0, rows0)
        load_e(ebuf0, se0, jnp.minimum(2 * i + 2, last_even))
        load_w(wbuf0, sw0, jnp.minimum(2 * i + 2, last_even))
        scatter(rows0, dstc0, ss0)
        gB.wait()
        pltpu.make_async_copy(wpack.at[c_base], wbuf1, sw1).wait()
        _copy_dst(ebuf1, dstc1)
        _scale_rows(wbuf1, rows1)
        load_e(ebuf1, se1, jnp.minimum(2 * i + 3, last_odd))
        load_w(wbuf1, sw1, jnp.minimum(2 * i + 3, last_odd))
        scatter(rows1, dstc1, ss1)
        return _

    lax.fori_loop(1, N_CHUNKS // 2, body, None)

    pltpu.make_async_copy(rows0, acc_buf.at[dstc0], ss0).wait()
    pltpu.make_async_copy(rows1, acc_buf.at[dstc1], ss1).wait()
    pltpu.make_async_copy(epack.at[c_base], ebuf0, se0).wait()
    pltpu.make_async_copy(epack.at[c_base], ebuf1, se1).wait()
    pltpu.make_async_copy(wpack.at[c_base], wbuf0, sw0).wait()
    pltpu.make_async_copy(wpack.at[c_base], wbuf1, sw1).wait()


def _combine_chunk(t0c, t1c, t2c, t3c, lowc, highc, coefs):
    c2l, c1l, c0l, c2h, c1h, c0h = coefs

    def body(j, _):
        for d in range(D_HALF // 16):
            sl = pl.ds(16 * d, 16)
            v0 = t0c[j, sl]
            v1 = t1c[j, sl]
            v2 = t2c[j, sl]
            v3 = t3c[j, sl]
            lowc[j, sl] = v3 + c2l * v2 + c1l * v1 + c0l * v0
            highc[j, sl] = v3 + c2h * v2 + c1h * v1 + c0h * v0
        return _

    lax.fori_loop(0, ROW_CHUNK, body, None)


def _filter_kernel(x_pair, epack, wpack, da_hbm,
                   low_hbm, high_hbm, tx1_hbm, tx2_hbm,
                   buf_a, buf_b,
                   ebuf0, ebuf1, wbuf0, wbuf1, rows0, rows1, dstc0, dstc1,
                   zbuf, t0c, t1c, t2c, t3c, lowc, highc, dav,
                   se0, se1, sw0, sw1, sg0, sg1, ss0, ss1):
    cid = lax.axis_index("c")
    sid = lax.axis_index("s")
    r0 = sid * ROWS_PER_TILE
    bufs = (ebuf0, ebuf1, wbuf0, wbuf1, rows0, rows1, dstc0, dstc1,
            se0, se1, sw0, sw1, sg0, sg1, ss0, ss1)

    pltpu.sync_copy(x_pair.at[cid, pl.ds(r0, ROWS_PER_TILE)],
                    buf_a.at[pl.ds(r0, ROWS_PER_TILE)])
    pltpu.sync_copy(da_hbm, dav)
    _zero_fill(zbuf, ROW_CHUNK)
    for q in range(N_ROW_CHUNKS):
        pltpu.sync_copy(zbuf, buf_b.at[pl.ds(r0 + q * ROW_CHUNK, ROW_CHUNK)])
    plsc.subcore_barrier()

    _hop(buf_a, buf_b, epack, wpack, sid, bufs)
    plsc.subcore_barrier()

    pltpu.sync_copy(buf_b.at[pl.ds(r0, ROWS_PER_TILE)],
                    tx1_hbm.at[cid, pl.ds(r0, ROWS_PER_TILE)])
    for q in range(N_ROW_CHUNKS):
        pltpu.sync_copy(zbuf, buf_a.at[pl.ds(r0 + q * ROW_CHUNK, ROW_CHUNK)])
    plsc.subcore_barrier()

    _hop(buf_b, buf_a, epack, wpack, sid, bufs)
    plsc.subcore_barrier()

    pltpu.sync_copy(buf_a.at[pl.ds(r0, ROWS_PER_TILE)],
                    tx2_hbm.at[cid, pl.ds(r0, ROWS_PER_TILE)])
    for q in range(N_ROW_CHUNKS):
        pltpu.sync_copy(zbuf, buf_b.at[pl.ds(r0 + q * ROW_CHUNK, ROW_CHUNK)])
    plsc.subcore_barrier()

    _hop(buf_a, buf_b, epack, wpack, sid, bufs)
    plsc.subcore_barrier()

    davec = dav[pl.ds(0, 16)]
    d = davec[0]
    av = davec[1]
    d2 = d * d
    c2l = -3.0 * d - av
    c1l = 3.0 * d2 + 2.0 * d * av
    c0l = -(d2 * d + d2 * av)
    c2h = -3.0 * d + av
    c1h = 3.0 * d2 - 2.0 * d * av
    c0h = d2 * av - d2 * d
    coefs = (c2l, c1l, c0l, c2h, c1h, c0h)

    for q in range(N_ROW_CHUNKS):
        rq = r0 + q * ROW_CHUNK
        pltpu.sync_copy(x_pair.at[cid, pl.ds(rq, ROW_CHUNK)], t0c)
        pltpu.sync_copy(tx1_hbm.at[cid, pl.ds(rq, ROW_CHUNK)], t1c)
        pltpu.sync_copy(tx2_hbm.at[cid, pl.ds(rq, ROW_CHUNK)], t2c)
        pltpu.sync_copy(buf_b.at[pl.ds(rq, ROW_CHUNK)], t3c)
        _combine_chunk(t0c, t1c, t2c, t3c, lowc, highc, coefs)
        pltpu.sync_copy(lowc, low_hbm.at[cid, pl.ds(rq, ROW_CHUNK)])
        pltpu.sync_copy(highc, high_hbm.at[cid, pl.ds(rq, ROW_CHUNK)])


@jax.jit
def _run(x_pair, epack, wpack, da):
    mesh = plsc.VectorSubcoreMesh(core_axis_name="c", subcore_axis_name="s")
    f = pl.kernel(
        _filter_kernel,
        mesh=mesh,
        compiler_params=pltpu.CompilerParams(use_tc_tiling_on_sc=False),
        out_type=[
            jax.ShapeDtypeStruct((2, N_PAD, D_HALF), jnp.float32),
            jax.ShapeDtypeStruct((2, N_PAD, D_HALF), jnp.float32),
            jax.ShapeDtypeStruct((2, N_PAD, D_HALF), jnp.float32),
            jax.ShapeDtypeStruct((2, N_PAD, D_HALF), jnp.float32),
        ],
        scratch_types=[
            pltpu.VMEM_SHARED((N_PAD, D_HALF), jnp.float32),
            pltpu.VMEM_SHARED((N_PAD, D_HALF), jnp.float32),
            pltpu.VMEM((2, CHUNK_E), jnp.int32),
            pltpu.VMEM((2, CHUNK_E), jnp.int32),
            pltpu.VMEM((CHUNK_E,), jnp.float32),
            pltpu.VMEM((CHUNK_E,), jnp.float32),
            pltpu.VMEM((CHUNK_E, D_HALF), jnp.float32),
            pltpu.VMEM((CHUNK_E, D_HALF), jnp.float32),
            pltpu.VMEM((CHUNK_E,), jnp.int32),
            pltpu.VMEM((CHUNK_E,), jnp.int32),
            pltpu.VMEM((ROW_CHUNK, D_HALF), jnp.float32),
            pltpu.VMEM((ROW_CHUNK, D_HALF), jnp.float32),
            pltpu.VMEM((ROW_CHUNK, D_HALF), jnp.float32),
            pltpu.VMEM((ROW_CHUNK, D_HALF), jnp.float32),
            pltpu.VMEM((ROW_CHUNK, D_HALF), jnp.float32),
            pltpu.VMEM((ROW_CHUNK, D_HALF), jnp.float32),
            pltpu.VMEM((ROW_CHUNK, D_HALF), jnp.float32),
            pltpu.VMEM((16,), jnp.float32),
            pltpu.SemaphoreType.DMA,
            pltpu.SemaphoreType.DMA,
            pltpu.SemaphoreType.DMA,
            pltpu.SemaphoreType.DMA,
            pltpu.SemaphoreType.DMA,
            pltpu.SemaphoreType.DMA,
            pltpu.SemaphoreType.DMA,
            pltpu.SemaphoreType.DMA,
        ],
    )
    return f(x_pair, epack, wpack, da)


def kernel(x, edge_index, edge_weight, delta, a):
    x_pair = x.reshape(N_NODES, 2, D_HALF).transpose(1, 0, 2)
    x_pair = jnp.pad(x_pair, ((0, 0), (0, N_PAD - N_NODES), (0, 0)))
    pad = E_PAD - N_EDGES
    src = jnp.pad(edge_index[1], (0, pad))
    dst = jnp.pad(edge_index[0], (0, pad))
    epack = jnp.stack([src, dst], axis=0)
    epack = epack.reshape(2, N_SUBCORES * N_CHUNKS, CHUNK_E).transpose(1, 0, 2)
    wpack = jnp.pad(edge_weight, (0, pad)).reshape(N_SUBCORES * N_CHUNKS, CHUNK_E)
    da = jnp.concatenate([delta, a, jnp.zeros((14,), jnp.float32)])
    low_p, high_p, _tx1, _tx2 = _run(x_pair, epack, wpack, da)
    low = low_p[:, :N_NODES].transpose(1, 0, 2).reshape(N_NODES, D_FEAT)
    high = high_p[:, :N_NODES].transpose(1, 0, 2).reshape(N_NODES, D_FEAT)
    return (low, high)
